# baseline (speedup 1.0000x reference)
import functools

import jax
import jax.numpy as jnp
from jax import lax
from jax.experimental import pallas as pl
from jax.experimental.pallas import tpu as pltpu
from jax.experimental.pallas import tpu_sc as plsc

B = 4
D = 256
N = 16384
S = 320
NC = 5
TK = 2048
NT = N // TK
UC = 8
CHUNK = 2048
NSTEP = CHUNK // 16


def _sc_keys_body(lab_hbm, idx_hbm, v_hbm, mx_hbm, lab_v, idx_v, v_v,
                  maxs_v, mxs_v, shared):
    c = lax.axis_index("c")
    s = lax.axis_index("s")
    g = c * 16 + s
    n_local = s // 8
    j = s % 8
    base = g * CHUNK

    pltpu.sync_copy(lab_hbm.at[pl.ds(base, CHUNK)], lab_v)
    pltpu.sync_copy(idx_hbm.at[pl.ds(base, CHUNK)], idx_v)

    def maxbody(i, acc):
        return jnp.maximum(acc, idx_v[pl.ds(i * 16, 16)])

    local_max = lax.fori_loop(0, NSTEP, maxbody, jnp.zeros((16,), jnp.int32))
    mxs_v[...] = local_max
    pltpu.sync_copy(mxs_v, shared.at[s])
    plsc.subcore_barrier()
    pltpu.sync_copy(shared.at[pl.ds(n_local * 8, 8)], maxs_v)
    acc = maxs_v[0]
    for r in range(1, 8):
        acc = jnp.maximum(acc, maxs_v[r])
    gdn = lax.GatherDimensionNumbers(
        offset_dims=(), collapsed_slice_dims=(0,), start_index_map=(0,)
    )
    for shift in (8, 4, 2, 1):
        perm = jnp.bitwise_xor(lax.iota(jnp.int32, 16), shift)
        shuf = lax.gather(
            acc, perm[:, None], dimension_numbers=gdn, slice_sizes=(1,),
            mode=lax.GatherScatterMode.PROMISE_IN_BOUNDS,
        )
        acc = jnp.maximum(acc, shuf)
    mxv = acc

    def vbody(i, carry):
        lab16 = lab_v[pl.ds(i * 16, 16)]
        idx16 = idx_v[pl.ds(i * 16, 16)]
        v_v[pl.ds(i * 16, 16)] = mxv * lab16 + idx16
        return carry

    lax.fori_loop(0, NSTEP, vbody, 0)
    pltpu.sync_copy(v_v, v_hbm.at[pl.ds(base, CHUNK)])

    @pl.when(j == 0)
    def _write_mx():
        mxs_v[...] = mxv
        pltpu.sync_copy(mxs_v, mx_hbm.at[c * 2 + n_local])


@functools.cache
def _sc_keys():
    mesh = plsc.VectorSubcoreMesh(core_axis_name="c", subcore_axis_name="s")
    return pl.kernel(
        _sc_keys_body,
        mesh=mesh,
        out_type=[
            jax.ShapeDtypeStruct((B * N,), jnp.int32),
            jax.ShapeDtypeStruct((B, 16), jnp.int32),
        ],
        scratch_types=[
            pltpu.VMEM((CHUNK,), jnp.int32),
            pltpu.VMEM((CHUNK,), jnp.int32),
            pltpu.VMEM((CHUNK,), jnp.int32),
            pltpu.VMEM((8, 16), jnp.int32),
            pltpu.VMEM((16,), jnp.int32),
            pltpu.VMEM_SHARED((16, 16), jnp.int32),
        ],
    )


def _sums_kernel(v_ref, feat_ref, sums_ref, counts_ref):
    tt = pl.program_id(1)
    v = v_ref[0]
    sidx = lax.broadcasted_iota(jnp.int32, (S, TK), 0)
    onehot = (sidx == v).astype(jnp.float32)
    feat = feat_ref[0]
    part = lax.dot_general(
        onehot, feat,
        dimension_numbers=(((1,), (1,)), ((), ())),
        preferred_element_type=jnp.float32,
    )
    cnt = jnp.sum(onehot, axis=1, keepdims=True)

    @pl.when(tt == 0)
    def _init():
        sums_ref[0] = part
        counts_ref[0] = cnt

    @pl.when(tt != 0)
    def _acc():
        sums_ref[0] += part
        counts_ref[0] += cnt


def _epilogue_kernel(sums_ref, counts_ref, mx_ref, out_ref, mean_s, m_s):
    blockones = (
        lax.broadcasted_iota(jnp.int32, (UC * D, UC), 0) // D
        == lax.broadcasted_iota(jnp.int32, (UC * D, UC), 1)
    ).astype(jnp.float32)
    total = jnp.float32(0.0)
    acc = jnp.float32(0.0)
    for n in range(B):
        cnt = counts_ref[n]
        mean_s[...] = sums_ref[n] / jnp.maximum(cnt, 1.0)
        nonempty = cnt > 0.0
        nseg = jnp.sum(nonempty.astype(jnp.float32))
        vv = lax.broadcasted_iota(jnp.int32, (S, 1), 0).astype(jnp.float32)
        vmax = jnp.max(jnp.where(nonempty, vv, -1.0))
        v2 = jnp.max(jnp.where(nonempty & (vv != vmax), vv, -1.0))
        prev_val = jnp.where(nseg >= 2.0, v2, vmax)
        mxf = mx_ref[n, 0].astype(jnp.float32)
        cls = jnp.ceil(vv / mxf - 1.0)
        last_cls = jnp.ceil(prev_val / mxf - 1.0)
        cls = jnp.where(vv == vmax, last_cls, cls)
        valid = (cnt >= 2.0) & (vv != 0.0) & (nseg > 1.0)
        cidx = lax.broadcasted_iota(jnp.int32, (S, NC), 1).astype(jnp.float32)
        m = (valid & (cls == cidx)).astype(jnp.float32)
        m_s[...] = m
        ks = jnp.sum(m, axis=0, keepdims=True)

        def body(uc, ss):
            chunk = mean_s[pl.ds(uc * UC, UC), :]
            mean = mean_s[...]
            cat = jnp.concatenate(
                [jnp.abs(mean - chunk[s : s + 1, :]) for s in range(UC)],
                axis=1,
            )
            pd_t = lax.dot_general(
                cat, blockones,
                dimension_numbers=(((1,), (0,)), ((), ())),
                preferred_element_type=jnp.float32,
            )
            r = lax.dot_general(
                pd_t, m_s[...],
                dimension_numbers=(((0,), (0,)), ((), ())),
                preferred_element_type=jnp.float32,
            )
            mu = m_s[pl.ds(uc * UC, UC), :]
            return ss + lax.dot_general(
                mu, r,
                dimension_numbers=(((0,), (0,)), ((), ())),
                preferred_element_type=jnp.float32,
            )

        ss = lax.fori_loop(0, S // UC, body, jnp.zeros((NC, NC), jnp.float32))

        for i in range(NC - 1):
            for j in range(i + 1, NC):
                npairs = ks[0, i] * ks[0, j]
                denom = jnp.maximum(npairs, 1.0) * jnp.float32(D)
                ret = ss[i, j] / denom
                ret = jnp.where(ret < 1.0, 0.5 * ret * ret, ret - 0.5)
                flag = (npairs > 0.0).astype(jnp.float32)
                total += flag
                acc += ret * flag

    mean_loss = acc / jnp.maximum(total, 1.0)
    loss = jnp.where(total > 0.0, -mean_loss, 0.0)
    loss = jnp.where(loss == 0.0, -jnp.float32(B), loss)
    out_ref[0, 0] = -jnp.log(-loss / jnp.float32(B))


def kernel(feature_out, labels, indexes):
    feat = feature_out.reshape(B, D, N)
    lab_flat = labels.reshape(B * N).astype(jnp.int32)
    idx_flat = indexes.reshape(B * N).astype(jnp.int32)

    v_flat, mx = _sc_keys()(lab_flat, idx_flat)
    v3 = v_flat.reshape(B, 1, N)

    sums, counts = pl.pallas_call(
        _sums_kernel,
        grid=(B, NT),
        in_specs=[
            pl.BlockSpec((1, 1, TK), lambda n, t: (n, 0, t)),
            pl.BlockSpec((1, D, TK), lambda n, t: (n, 0, t)),
        ],
        out_specs=[
            pl.BlockSpec((1, S, D), lambda n, t: (n, 0, 0)),
            pl.BlockSpec((1, S, 1), lambda n, t: (n, 0, 0)),
        ],
        out_shape=[
            jax.ShapeDtypeStruct((B, S, D), jnp.float32),
            jax.ShapeDtypeStruct((B, S, 1), jnp.float32),
        ],
    )(v3, feat)

    out = pl.pallas_call(
        _epilogue_kernel,
        in_specs=[
            pl.BlockSpec(memory_space=pltpu.VMEM),
            pl.BlockSpec(memory_space=pltpu.VMEM),
            pl.BlockSpec(memory_space=pltpu.SMEM),
        ],
        out_specs=pl.BlockSpec(memory_space=pltpu.SMEM),
        out_shape=jax.ShapeDtypeStruct((1, 1), jnp.float32),
        scratch_shapes=[
            pltpu.VMEM((S, D), jnp.float32),
            pltpu.VMEM((S, NC), jnp.float32),
        ],
    )(sums, counts, mx)
    return out.reshape(1)
